# baseline (speedup 1.0000x reference)
import functools

import jax
import jax.numpy as jnp
from jax.experimental import pallas as pl
from jax.experimental.pallas import tpu as pltpu

B = 4
S = 512
H = 256
K = 16
G_HID = 256
NEG_BIG = -1e30
F32 = jnp.float32


def _gates(z, c):
    i = jax.nn.sigmoid(z[:, 0 * H:1 * H])
    f = jax.nn.sigmoid(z[:, 1 * H:2 * H])
    g = jnp.tanh(z[:, 2 * H:3 * H])
    o = jax.nn.sigmoid(z[:, 3 * H:4 * H])
    c2 = f * c + i * g
    h2 = o * jnp.tanh(c2)
    return h2, c2


def _lstm_kernel(pos_ref, wpT_ref, bp_ref,
                 w0fT_ref, u0fT_ref, b0f_ref,
                 w0bT_ref, u0bT_ref, b0b_ref,
                 w1faT_ref, w1fbT_ref, u1fT_ref, b1f_ref,
                 w1baT_ref, w1bbT_ref, u1bT_ref, b1b_ref,
                 feats_ref,
                 xf_ref, xb_ref, hf_ref, hb_ref):
    x = jnp.dot(pos_ref[...], wpT_ref[...], preferred_element_type=F32)
    x = x + bp_ref[...]

    xf_ref[...] = jnp.dot(x, w0fT_ref[...], preferred_element_type=F32) + b0f_ref[...]
    xb_ref[...] = jnp.dot(x, w0bT_ref[...], preferred_element_type=F32) + b0b_ref[...]

    u0fT = u0fT_ref[...]
    u0bT = u0bT_ref[...]

    def step0(i, carry):
        hf, cf, hb, cb = carry
        xf8 = xf_ref[pl.ds(i * 2 * B, 2 * B), :]
        zfa = xf8[0:B] + jnp.dot(hf, u0fT, preferred_element_type=F32)
        hfa, cf = _gates(zfa, cf)
        zfb = xf8[B:2 * B] + jnp.dot(hfa, u0fT, preferred_element_type=F32)
        hf, cf = _gates(zfb, cf)
        hf_ref[pl.ds(i * 2 * B, 2 * B), :] = jnp.concatenate([hfa, hf], axis=0)
        sb = (S // 2 - 1) - i
        xb8 = xb_ref[pl.ds(sb * 2 * B, 2 * B), :]
        zbb = xb8[B:2 * B] + jnp.dot(hb, u0bT, preferred_element_type=F32)
        hbb, cb = _gates(zbb, cb)
        zba = xb8[0:B] + jnp.dot(hbb, u0bT, preferred_element_type=F32)
        hb, cb = _gates(zba, cb)
        hb_ref[pl.ds(sb * 2 * B, 2 * B), :] = jnp.concatenate([hb, hbb], axis=0)
        return hf, cf, hb, cb

    z4 = jnp.zeros((B, H), F32)
    jax.lax.fori_loop(0, S // 2, step0, (z4, z4, z4, z4))

    hf0 = hf_ref[...]
    hb0 = hb_ref[...]
    xf_ref[...] = (jnp.dot(hf0, w1faT_ref[...], preferred_element_type=F32)
                   + jnp.dot(hb0, w1fbT_ref[...], preferred_element_type=F32)
                   + b1f_ref[...])
    xb_ref[...] = (jnp.dot(hf0, w1baT_ref[...], preferred_element_type=F32)
                   + jnp.dot(hb0, w1bbT_ref[...], preferred_element_type=F32)
                   + b1b_ref[...])

    u1fT = u1fT_ref[...]
    u1bT = u1bT_ref[...]

    def step1(i, carry):
        hf, cf, hb, cb = carry
        xf8 = xf_ref[pl.ds(i * 2 * B, 2 * B), :]
        zfa = xf8[0:B] + jnp.dot(hf, u1fT, preferred_element_type=F32)
        hfa, cf = _gates(zfa, cf)
        zfb = xf8[B:2 * B] + jnp.dot(hfa, u1fT, preferred_element_type=F32)
        hf, cf = _gates(zfb, cf)
        feats_ref[pl.ds(i * 2 * B, 2 * B), 0:H] = jnp.concatenate([hfa, hf], axis=0)
        sb = (S // 2 - 1) - i
        xb8 = xb_ref[pl.ds(sb * 2 * B, 2 * B), :]
        zbb = xb8[B:2 * B] + jnp.dot(hb, u1bT, preferred_element_type=F32)
        hbb, cb = _gates(zbb, cb)
        zba = xb8[0:B] + jnp.dot(hbb, u1bT, preferred_element_type=F32)
        hb, cb = _gates(zba, cb)
        feats_ref[pl.ds(sb * 2 * B, 2 * B), H:2 * H] = jnp.concatenate([hb, hbb], axis=0)
        return hf, cf, hb, cb

    jax.lax.fori_loop(0, S // 2, step1, (z4, z4, z4, z4))


def _run_lstm(pos_tm, params):
    p = params
    l0, l1 = p['lstm'][0], p['lstm'][1]
    ins = [
        pos_tm,
        p['W_proj'].T,
        p['b_proj'][None, :],
        l0['Wih_f'].T, l0['Whh_f'].T, (l0['bih_f'] + l0['bhh_f'])[None, :],
        l0['Wih_b'].T, l0['Whh_b'].T, (l0['bih_b'] + l0['bhh_b'])[None, :],
        l1['Wih_f'][:, :H].T, l1['Wih_f'][:, H:].T, l1['Whh_f'].T,
        (l1['bih_f'] + l1['bhh_f'])[None, :],
        l1['Wih_b'][:, :H].T, l1['Wih_b'][:, H:].T, l1['Whh_b'].T,
        (l1['bih_b'] + l1['bhh_b'])[None, :],
    ]
    feats = pl.pallas_call(
        _lstm_kernel,
        out_shape=jax.ShapeDtypeStruct((S * B, 2 * H), F32),
        scratch_shapes=[
            pltpu.VMEM((S * B, 4 * H), F32),
            pltpu.VMEM((S * B, 4 * H), F32),
            pltpu.VMEM((S * B, H), F32),
            pltpu.VMEM((S * B, H), F32),
        ],
    )(*ins)
    return feats


def _edge_kernel(g_ref, gT_ref, x2c_ref, x2r_ref, w1T_ref, b1_ref, w2T_ref,
                 b2_ref, *refs, d_out, last, head_ws=None):
    out_refs, ohs_ref = refs[:-1], refs[-1]
    gb = g_ref[0]
    gbT = gT_ref[0]
    prod = jnp.dot(gb, gbT, preferred_element_type=F32)
    dist = (x2c_ref[0] + x2r_ref[0]) - 2.0 * prod

    w1T = w1T_ref[...]
    w2T = w2T_ref[...]
    b1 = b1_ref[...]
    dpad = gb.shape[1]

    g1 = gb.astype(jnp.bfloat16).astype(F32)
    r = gb - g1
    g2 = r.astype(jnp.bfloat16).astype(F32)
    g3 = r - g2

    acc0 = jnp.dot(gb, w1T[:dpad], preferred_element_type=F32)
    nchunk = (dpad + 255) // 256

    iota = jax.lax.broadcasted_iota(jnp.int32, (S, S), 1)
    for k in range(K):
        m = jnp.min(dist, axis=1, keepdims=True)
        eq = dist <= m
        aidx = jnp.min(jnp.where(eq, iota, S), axis=1, keepdims=True)
        onehot = iota == aidx
        dist = jnp.where(onehot, jnp.inf, dist)
        ohs_ref[k] = onehot.astype(F32)
    out = jnp.full((S, d_out), NEG_BIG, F32)
    for k in range(K):
        oh = ohs_ref[k]
        xj = (jnp.dot(oh, g1, preferred_element_type=F32)
              + jnp.dot(oh, g2, preferred_element_type=F32)
              + jnp.dot(oh, g3, preferred_element_type=F32))
        dlt = xj - gb
        acc = acc0
        for c in range(nchunk):
            lo = c * 256
            hi = min(lo + 256, dpad)
            acc = acc + jnp.dot(dlt[:, lo:hi], w1T[dpad + lo:dpad + hi],
                                preferred_element_type=F32)
        h = jnp.maximum(acc + b1, 0.0)
        o = jnp.dot(h, w2T, preferred_element_type=F32)
        out = jnp.maximum(out, o)
    out = out + b2_ref[...]
    if not last:
        out_refs[0][0] = jnp.maximum(out, 0.0)
        return
    wposT_ref, bpos_ref, wquatT_ref, bquat_ref, wscaleT_ref, bscale_ref = head_ws
    dp = jnp.dot(out, wposT_ref[...], preferred_element_type=F32) + bpos_ref[...]
    dq = jnp.dot(out, wquatT_ref[...], preferred_element_type=F32) + bquat_ref[...]
    ds = jnp.dot(out, wscaleT_ref[...], preferred_element_type=F32) + bscale_ref[...]
    nrm = jnp.sqrt(jnp.sum(dq * dq, axis=1, keepdims=True))
    dq = dq / jnp.maximum(nrm, 1e-12)
    out_refs[0][0] = dp
    out_refs[1][0] = dq
    out_refs[2][0] = ds


def _run_edge(g, gl, last, head_params=None):
    d = g.shape[-1]
    d_out = gl['W2'].shape[0]
    x2 = jnp.sum(g * g, axis=-1)
    w1T = gl['W1'].T
    if d % 128:
        dp = d + (-d % 128)
        g = jnp.pad(g, ((0, 0), (0, 0), (0, dp - d)))
        zrows = jnp.zeros((dp - d, w1T.shape[1]), F32)
        w1T = jnp.concatenate([w1T[:d], zrows, w1T[d:], zrows], axis=0)
        d = dp
    gT = jnp.transpose(g, (0, 2, 1))
    ins = [g, gT, x2[:, :, None], x2[:, None, :],
           w1T, gl['b1'][None, :], gl['W2'].T, gl['b2'][None, :]]
    in_specs = [
        pl.BlockSpec((1, S, d), lambda b: (b, 0, 0)),
        pl.BlockSpec((1, d, S), lambda b: (b, 0, 0)),
        pl.BlockSpec((1, S, 1), lambda b: (b, 0, 0)),
        pl.BlockSpec((1, 1, S), lambda b: (b, 0, 0)),
        pl.BlockSpec((2 * d, G_HID), lambda b: (0, 0)),
        pl.BlockSpec((1, G_HID), lambda b: (0, 0)),
        pl.BlockSpec((G_HID, d_out), lambda b: (0, 0)),
        pl.BlockSpec((1, d_out), lambda b: (0, 0)),
    ]
    if not last:
        kfn = functools.partial(_edge_kernel, d_out=d_out, last=False)
        out = pl.pallas_call(
            kfn,
            grid=(B,),
            in_specs=in_specs,
            out_specs=pl.BlockSpec((1, S, d_out), lambda b: (b, 0, 0)),
            out_shape=jax.ShapeDtypeStruct((B, S, d_out), F32),
            scratch_shapes=[pltpu.VMEM((K, S, S), F32)],
            compiler_params=pltpu.CompilerParams(
                dimension_semantics=("arbitrary",)),
        )(*ins)
        return out
    hp = head_params
    ins += [hp['W_pos'].T, hp['b_pos'][None, :],
            hp['W_quat'].T, hp['b_quat'][None, :],
            hp['W_scale'].T, hp['b_scale'][None, :]]
    in_specs += [
        pl.BlockSpec((d_out, 3), lambda b: (0, 0)),
        pl.BlockSpec((1, 3), lambda b: (0, 0)),
        pl.BlockSpec((d_out, 4), lambda b: (0, 0)),
        pl.BlockSpec((1, 4), lambda b: (0, 0)),
        pl.BlockSpec((d_out, 3), lambda b: (0, 0)),
        pl.BlockSpec((1, 3), lambda b: (0, 0)),
    ]

    def kfn(g_ref, gT_ref, x2c_ref, x2r_ref, w1T_ref, b1_ref, w2T_ref, b2_ref,
            wposT, bpos, wquatT, bquat, wscaleT, bscale,
            dp_ref, dq_ref, ds_ref, ohs_ref):
        _edge_kernel(g_ref, gT_ref, x2c_ref, x2r_ref, w1T_ref, b1_ref,
                     w2T_ref, b2_ref, dp_ref, dq_ref, ds_ref, ohs_ref,
                     d_out=d_out, last=True,
                     head_ws=(wposT, bpos, wquatT, bquat, wscaleT, bscale))

    dp, dq, ds = pl.pallas_call(
        kfn,
        grid=(B,),
        in_specs=in_specs,
        out_specs=[
            pl.BlockSpec((1, S, 3), lambda b: (b, 0, 0)),
            pl.BlockSpec((1, S, 4), lambda b: (b, 0, 0)),
            pl.BlockSpec((1, S, 3), lambda b: (b, 0, 0)),
        ],
        out_shape=[
            jax.ShapeDtypeStruct((B, S, 3), F32),
            jax.ShapeDtypeStruct((B, S, 4), F32),
            jax.ShapeDtypeStruct((B, S, 3), F32),
        ],
        scratch_shapes=[pltpu.VMEM((K, S, S), F32)],
        compiler_params=pltpu.CompilerParams(
            dimension_semantics=("arbitrary",)),
    )(*ins)
    return dp, dq, ds


def kernel(positions, batch_indices, params):
    del batch_indices
    pos_tm = jnp.transpose(positions, (1, 0, 2)).reshape(S * B, 3)
    feats_tm = _run_lstm(pos_tm, params)
    feats = jnp.transpose(feats_tm.reshape(S, B, 2 * H), (1, 0, 2))
    g = jnp.concatenate([positions, feats], axis=-1)
    glayers = params['graph']
    g = _run_edge(g, glayers[0], last=False)
    g = _run_edge(g, glayers[1], last=False)
    return _run_edge(g, glayers[2], last=True, head_params=params)


# speedup vs baseline: 1.0788x; 1.0788x over previous
import functools

import jax
import jax.numpy as jnp
from jax.experimental import pallas as pl
from jax.experimental.pallas import tpu as pltpu

B = 4
S = 512
H = 256
K = 16
G_HID = 256
NEG_BIG = -1e30
F32 = jnp.float32


def _gates(z, c):
    i = jax.nn.sigmoid(z[:, 0 * H:1 * H])
    f = jax.nn.sigmoid(z[:, 1 * H:2 * H])
    g = jnp.tanh(z[:, 2 * H:3 * H])
    o = jax.nn.sigmoid(z[:, 3 * H:4 * H])
    c2 = f * c + i * g
    h2 = o * jnp.tanh(c2)
    return h2, c2


def _lstm_kernel(pos_ref, wpT_ref, bp_ref,
                 w0fT_ref, w0bT_ref, b0f_ref, b0b_ref, u0_ref,
                 w1faT_ref, w1fbT_ref, w1baT_ref, w1bbT_ref,
                 b1f_ref, b1b_ref, u1_ref,
                 feats_ref,
                 xf_ref, xb_ref, hf_ref, hb_ref):
    x = jnp.dot(pos_ref[...], wpT_ref[...], preferred_element_type=F32)
    x = x + bp_ref[...]

    xf_ref[...] = jnp.dot(x, w0fT_ref[...], preferred_element_type=F32) + b0f_ref[...]
    xb_ref[...] = jnp.dot(x, w0bT_ref[...], preferred_element_type=F32) + b0b_ref[...]

    si = jax.lax.broadcasted_iota(jnp.int32, (2 * B, H), 0)
    mtop = (si < B).astype(F32)
    mbot = 1.0 - mtop

    def make_step(u, store):
        def step(i, carry):
            hc, cc = carry
            xf2 = xf_ref[pl.ds(i * 2 * B, 2 * B), :]
            xb2 = xb_ref[pl.ds((S // 2 - 1 - i) * 2 * B, 2 * B), :]
            xa = jnp.concatenate([xf2[0:B], xb2[B:2 * B]], axis=0)
            hm = jnp.concatenate([hc * mtop, hc * mbot], axis=1)
            za = xa + jnp.dot(hm, u, preferred_element_type=F32)
            ha, ca = _gates(za, cc)
            xbp = jnp.concatenate([xf2[B:2 * B], xb2[0:B]], axis=0)
            hm2 = jnp.concatenate([ha * mtop, ha * mbot], axis=1)
            zb = xbp + jnp.dot(hm2, u, preferred_element_type=F32)
            hb2, cb2 = _gates(zb, ca)
            store(i, ha, hb2)
            return hb2, cb2
        return step

    def store0(i, ha, hb2):
        hf_ref[pl.ds(i * 2 * B, 2 * B), :] = jnp.concatenate(
            [ha[0:B], hb2[0:B]], axis=0)
        hb_ref[pl.ds((S // 2 - 1 - i) * 2 * B, 2 * B), :] = jnp.concatenate(
            [hb2[B:2 * B], ha[B:2 * B]], axis=0)

    z8 = jnp.zeros((2 * B, H), F32)
    jax.lax.fori_loop(0, S // 2, make_step(u0_ref[...], store0), (z8, z8))

    hf0 = hf_ref[...]
    hb0 = hb_ref[...]
    xf_ref[...] = (jnp.dot(hf0, w1faT_ref[...], preferred_element_type=F32)
                   + jnp.dot(hb0, w1fbT_ref[...], preferred_element_type=F32)
                   + b1f_ref[...])
    xb_ref[...] = (jnp.dot(hf0, w1baT_ref[...], preferred_element_type=F32)
                   + jnp.dot(hb0, w1bbT_ref[...], preferred_element_type=F32)
                   + b1b_ref[...])

    def store1(i, ha, hb2):
        feats_ref[pl.ds(i * 2 * B, 2 * B), 0:H] = jnp.concatenate(
            [ha[0:B], hb2[0:B]], axis=0)
        feats_ref[pl.ds((S // 2 - 1 - i) * 2 * B, 2 * B), H:2 * H] = jnp.concatenate(
            [hb2[B:2 * B], ha[B:2 * B]], axis=0)

    jax.lax.fori_loop(0, S // 2, make_step(u1_ref[...], store1), (z8, z8))


def _run_lstm(pos_tm, params):
    p = params
    l0, l1 = p['lstm'][0], p['lstm'][1]
    ins = [
        pos_tm,
        p['W_proj'].T,
        p['b_proj'][None, :],
        l0['Wih_f'].T, l0['Wih_b'].T,
        (l0['bih_f'] + l0['bhh_f'])[None, :],
        (l0['bih_b'] + l0['bhh_b'])[None, :],
        jnp.concatenate([l0['Whh_f'].T, l0['Whh_b'].T], axis=0),
        l1['Wih_f'][:, :H].T, l1['Wih_f'][:, H:].T,
        l1['Wih_b'][:, :H].T, l1['Wih_b'][:, H:].T,
        (l1['bih_f'] + l1['bhh_f'])[None, :],
        (l1['bih_b'] + l1['bhh_b'])[None, :],
        jnp.concatenate([l1['Whh_f'].T, l1['Whh_b'].T], axis=0),
    ]
    feats = pl.pallas_call(
        _lstm_kernel,
        out_shape=jax.ShapeDtypeStruct((S * B, 2 * H), F32),
        scratch_shapes=[
            pltpu.VMEM((S * B, 4 * H), F32),
            pltpu.VMEM((S * B, 4 * H), F32),
            pltpu.VMEM((S * B, H), F32),
            pltpu.VMEM((S * B, H), F32),
        ],
    )(*ins)
    return feats


def _edge_kernel(g_ref, gT_ref, x2c_ref, x2r_ref, w1T_ref, b1_ref, w2T_ref,
                 b2_ref, *refs, d_out, last, head_ws=None):
    out_refs, ohs_ref = refs[:-1], refs[-1]
    gb = g_ref[0]
    gbT = gT_ref[0]
    prod = jnp.dot(gb, gbT, preferred_element_type=F32)
    dist = (x2c_ref[0] + x2r_ref[0]) - 2.0 * prod

    w1T = w1T_ref[...]
    w2T = w2T_ref[...]
    b1 = b1_ref[...]
    dpad = gb.shape[1]

    g1 = gb.astype(jnp.bfloat16).astype(F32)
    r = gb - g1
    g2 = r.astype(jnp.bfloat16).astype(F32)
    g3 = r - g2

    acc0 = jnp.dot(gb, w1T[:dpad], preferred_element_type=F32)
    nchunk = (dpad + 255) // 256

    iota = jax.lax.broadcasted_iota(jnp.int32, (S, S), 1)
    for k in range(K):
        m = jnp.min(dist, axis=1, keepdims=True)
        eq = dist <= m
        aidx = jnp.min(jnp.where(eq, iota, S), axis=1, keepdims=True)
        onehot = iota == aidx
        dist = jnp.where(onehot, jnp.inf, dist)
        ohs_ref[k] = onehot.astype(F32)
    out = jnp.full((S, d_out), NEG_BIG, F32)
    for k in range(K):
        oh = ohs_ref[k]
        xj = (jnp.dot(oh, g1, preferred_element_type=F32)
              + jnp.dot(oh, g2, preferred_element_type=F32)
              + jnp.dot(oh, g3, preferred_element_type=F32))
        dlt = xj - gb
        acc = acc0
        for c in range(nchunk):
            lo = c * 256
            hi = min(lo + 256, dpad)
            acc = acc + jnp.dot(dlt[:, lo:hi], w1T[dpad + lo:dpad + hi],
                                preferred_element_type=F32)
        h = jnp.maximum(acc + b1, 0.0)
        o = jnp.dot(h, w2T, preferred_element_type=F32)
        out = jnp.maximum(out, o)
    out = out + b2_ref[...]
    if not last:
        out_refs[0][0] = jnp.maximum(out, 0.0)
        return
    wposT_ref, bpos_ref, wquatT_ref, bquat_ref, wscaleT_ref, bscale_ref = head_ws
    dp = jnp.dot(out, wposT_ref[...], preferred_element_type=F32) + bpos_ref[...]
    dq = jnp.dot(out, wquatT_ref[...], preferred_element_type=F32) + bquat_ref[...]
    ds = jnp.dot(out, wscaleT_ref[...], preferred_element_type=F32) + bscale_ref[...]
    nrm = jnp.sqrt(jnp.sum(dq * dq, axis=1, keepdims=True))
    dq = dq / jnp.maximum(nrm, 1e-12)
    out_refs[0][0] = dp
    out_refs[1][0] = dq
    out_refs[2][0] = ds


def _run_edge(g, gl, last, head_params=None):
    d = g.shape[-1]
    d_out = gl['W2'].shape[0]
    x2 = jnp.sum(g * g, axis=-1)
    w1T = gl['W1'].T
    if d % 128:
        dp = d + (-d % 128)
        g = jnp.pad(g, ((0, 0), (0, 0), (0, dp - d)))
        zrows = jnp.zeros((dp - d, w1T.shape[1]), F32)
        w1T = jnp.concatenate([w1T[:d], zrows, w1T[d:], zrows], axis=0)
        d = dp
    gT = jnp.transpose(g, (0, 2, 1))
    ins = [g, gT, x2[:, :, None], x2[:, None, :],
           w1T, gl['b1'][None, :], gl['W2'].T, gl['b2'][None, :]]
    in_specs = [
        pl.BlockSpec((1, S, d), lambda b: (b, 0, 0)),
        pl.BlockSpec((1, d, S), lambda b: (b, 0, 0)),
        pl.BlockSpec((1, S, 1), lambda b: (b, 0, 0)),
        pl.BlockSpec((1, 1, S), lambda b: (b, 0, 0)),
        pl.BlockSpec((2 * d, G_HID), lambda b: (0, 0)),
        pl.BlockSpec((1, G_HID), lambda b: (0, 0)),
        pl.BlockSpec((G_HID, d_out), lambda b: (0, 0)),
        pl.BlockSpec((1, d_out), lambda b: (0, 0)),
    ]
    if not last:
        kfn = functools.partial(_edge_kernel, d_out=d_out, last=False)
        out = pl.pallas_call(
            kfn,
            grid=(B,),
            in_specs=in_specs,
            out_specs=pl.BlockSpec((1, S, d_out), lambda b: (b, 0, 0)),
            out_shape=jax.ShapeDtypeStruct((B, S, d_out), F32),
            scratch_shapes=[pltpu.VMEM((K, S, S), F32)],
            compiler_params=pltpu.CompilerParams(
                dimension_semantics=("arbitrary",)),
        )(*ins)
        return out
    hp = head_params
    ins += [hp['W_pos'].T, hp['b_pos'][None, :],
            hp['W_quat'].T, hp['b_quat'][None, :],
            hp['W_scale'].T, hp['b_scale'][None, :]]
    in_specs += [
        pl.BlockSpec((d_out, 3), lambda b: (0, 0)),
        pl.BlockSpec((1, 3), lambda b: (0, 0)),
        pl.BlockSpec((d_out, 4), lambda b: (0, 0)),
        pl.BlockSpec((1, 4), lambda b: (0, 0)),
        pl.BlockSpec((d_out, 3), lambda b: (0, 0)),
        pl.BlockSpec((1, 3), lambda b: (0, 0)),
    ]

    def kfn(g_ref, gT_ref, x2c_ref, x2r_ref, w1T_ref, b1_ref, w2T_ref, b2_ref,
            wposT, bpos, wquatT, bquat, wscaleT, bscale,
            dp_ref, dq_ref, ds_ref, ohs_ref):
        _edge_kernel(g_ref, gT_ref, x2c_ref, x2r_ref, w1T_ref, b1_ref,
                     w2T_ref, b2_ref, dp_ref, dq_ref, ds_ref, ohs_ref,
                     d_out=d_out, last=True,
                     head_ws=(wposT, bpos, wquatT, bquat, wscaleT, bscale))

    dp, dq, ds = pl.pallas_call(
        kfn,
        grid=(B,),
        in_specs=in_specs,
        out_specs=[
            pl.BlockSpec((1, S, 3), lambda b: (b, 0, 0)),
            pl.BlockSpec((1, S, 4), lambda b: (b, 0, 0)),
            pl.BlockSpec((1, S, 3), lambda b: (b, 0, 0)),
        ],
        out_shape=[
            jax.ShapeDtypeStruct((B, S, 3), F32),
            jax.ShapeDtypeStruct((B, S, 4), F32),
            jax.ShapeDtypeStruct((B, S, 3), F32),
        ],
        scratch_shapes=[pltpu.VMEM((K, S, S), F32)],
        compiler_params=pltpu.CompilerParams(
            dimension_semantics=("arbitrary",)),
    )(*ins)
    return dp, dq, ds


def kernel(positions, batch_indices, params):
    del batch_indices
    pos_tm = jnp.transpose(positions, (1, 0, 2)).reshape(S * B, 3)
    feats_tm = _run_lstm(pos_tm, params)
    feats = jnp.transpose(feats_tm.reshape(S, B, 2 * H), (1, 0, 2))
    g = jnp.concatenate([positions, feats], axis=-1)
    glayers = params['graph']
    g = _run_edge(g, glayers[0], last=False)
    g = _run_edge(g, glayers[1], last=False)
    return _run_edge(g, glayers[2], last=True, head_params=params)


# speedup vs baseline: 1.1013x; 1.0209x over previous
import functools

import jax
import jax.numpy as jnp
from jax.experimental import pallas as pl
from jax.experimental.pallas import tpu as pltpu

B = 4
S = 512
H = 256
K = 16
G_HID = 256
NEG_BIG = -1e30
F32 = jnp.float32


def _gates(z, c):
    i = jax.nn.sigmoid(z[:, 0 * H:1 * H])
    f = jax.nn.sigmoid(z[:, 1 * H:2 * H])
    g = jnp.tanh(z[:, 2 * H:3 * H])
    o = jax.nn.sigmoid(z[:, 3 * H:4 * H])
    c2 = f * c + i * g
    h2 = o * jnp.tanh(c2)
    return h2, c2


def _lstm_kernel(pos_ref, wpT_ref, bp_ref,
                 w0fT_ref, w0bT_ref, b0f_ref, b0b_ref, u0_ref,
                 w1faT_ref, w1fbT_ref, w1baT_ref, w1bbT_ref,
                 b1f_ref, b1b_ref, u1_ref,
                 feats_ref,
                 xf_ref, xb_ref, hf_ref, hb_ref):
    x = jnp.dot(pos_ref[...], wpT_ref[...], preferred_element_type=F32)
    x = x + bp_ref[...]

    xf_ref[...] = jnp.dot(x, w0fT_ref[...], preferred_element_type=F32) + b0f_ref[...]
    xb_ref[...] = jnp.dot(x, w0bT_ref[...], preferred_element_type=F32) + b0b_ref[...]

    si = jax.lax.broadcasted_iota(jnp.int32, (2 * B, H), 0)
    mtop = (si < B).astype(F32)
    mbot = 1.0 - mtop

    def make_step(u, store):
        def step(i, carry):
            hc, cc = carry
            xf2 = xf_ref[pl.ds(i * 2 * B, 2 * B), :]
            xb2 = xb_ref[pl.ds((S // 2 - 1 - i) * 2 * B, 2 * B), :]
            xa = jnp.concatenate([xf2[0:B], xb2[B:2 * B]], axis=0)
            hm = jnp.concatenate([hc * mtop, hc * mbot], axis=1)
            za = xa + jnp.dot(hm, u, preferred_element_type=F32)
            ha, ca = _gates(za, cc)
            xbp = jnp.concatenate([xf2[B:2 * B], xb2[0:B]], axis=0)
            hm2 = jnp.concatenate([ha * mtop, ha * mbot], axis=1)
            zb = xbp + jnp.dot(hm2, u, preferred_element_type=F32)
            hb2, cb2 = _gates(zb, ca)
            store(i, ha, hb2)
            return hb2, cb2
        return step

    def store0(i, ha, hb2):
        hf_ref[pl.ds(i * 2 * B, 2 * B), :] = jnp.concatenate(
            [ha[0:B], hb2[0:B]], axis=0)
        hb_ref[pl.ds((S // 2 - 1 - i) * 2 * B, 2 * B), :] = jnp.concatenate(
            [hb2[B:2 * B], ha[B:2 * B]], axis=0)

    z8 = jnp.zeros((2 * B, H), F32)
    jax.lax.fori_loop(0, S // 2, make_step(u0_ref[...], store0), (z8, z8))

    hf0 = hf_ref[...]
    hb0 = hb_ref[...]
    xf_ref[...] = (jnp.dot(hf0, w1faT_ref[...], preferred_element_type=F32)
                   + jnp.dot(hb0, w1fbT_ref[...], preferred_element_type=F32)
                   + b1f_ref[...])
    xb_ref[...] = (jnp.dot(hf0, w1baT_ref[...], preferred_element_type=F32)
                   + jnp.dot(hb0, w1bbT_ref[...], preferred_element_type=F32)
                   + b1b_ref[...])

    def store1(i, ha, hb2):
        feats_ref[pl.ds(i * 2 * B, 2 * B), 0:H] = jnp.concatenate(
            [ha[0:B], hb2[0:B]], axis=0)
        feats_ref[pl.ds((S // 2 - 1 - i) * 2 * B, 2 * B), H:2 * H] = jnp.concatenate(
            [hb2[B:2 * B], ha[B:2 * B]], axis=0)

    jax.lax.fori_loop(0, S // 2, make_step(u1_ref[...], store1), (z8, z8))


def _run_lstm(pos_tm, params):
    p = params
    l0, l1 = p['lstm'][0], p['lstm'][1]
    ins = [
        pos_tm,
        p['W_proj'].T,
        p['b_proj'][None, :],
        l0['Wih_f'].T, l0['Wih_b'].T,
        (l0['bih_f'] + l0['bhh_f'])[None, :],
        (l0['bih_b'] + l0['bhh_b'])[None, :],
        jnp.concatenate([l0['Whh_f'].T, l0['Whh_b'].T], axis=0),
        l1['Wih_f'][:, :H].T, l1['Wih_f'][:, H:].T,
        l1['Wih_b'][:, :H].T, l1['Wih_b'][:, H:].T,
        (l1['bih_f'] + l1['bhh_f'])[None, :],
        (l1['bih_b'] + l1['bhh_b'])[None, :],
        jnp.concatenate([l1['Whh_f'].T, l1['Whh_b'].T], axis=0),
    ]
    feats = pl.pallas_call(
        _lstm_kernel,
        out_shape=jax.ShapeDtypeStruct((S * B, 2 * H), F32),
        scratch_shapes=[
            pltpu.VMEM((S * B, 4 * H), F32),
            pltpu.VMEM((S * B, 4 * H), F32),
            pltpu.VMEM((S * B, H), F32),
            pltpu.VMEM((S * B, H), F32),
        ],
    )(*ins)
    return feats


def _edge_kernel(g_ref, gT_ref, x2c_ref, x2r_ref, w1T_ref, b1_ref, w2T_ref,
                 b2_ref, *refs, d_out, last, head_ws=None):
    out_refs, ohs_ref = refs[:-1], refs[-1]
    gb = g_ref[0]
    gbT = gT_ref[0]
    prod = jnp.dot(gb, gbT, preferred_element_type=F32)
    dist = (x2c_ref[0] + x2r_ref[0]) - 2.0 * prod

    w1T = w1T_ref[...]
    w2T = w2T_ref[...]
    b1 = b1_ref[...]
    dpad = gb.shape[1]

    g1 = gb.astype(jnp.bfloat16).astype(F32)
    r = gb - g1
    g2 = r.astype(jnp.bfloat16).astype(F32)
    g3 = r - g2

    acc0 = jnp.dot(gb, w1T[:dpad], preferred_element_type=F32)
    nchunk = (dpad + 255) // 256

    iota = jax.lax.broadcasted_iota(jnp.int32, (S, S), 1)
    for k in range(K):
        aidx = jnp.argmin(dist, axis=1)[:, None]
        onehot = iota == aidx
        dist = jnp.where(onehot, jnp.inf, dist)
        ohs_ref[k] = onehot.astype(F32)
    out = jnp.full((S, d_out), NEG_BIG, F32)
    for k in range(K):
        oh = ohs_ref[k]
        xj = (jnp.dot(oh, g1, preferred_element_type=F32)
              + jnp.dot(oh, g2, preferred_element_type=F32)
              + jnp.dot(oh, g3, preferred_element_type=F32))
        dlt = xj - gb
        acc = acc0
        for c in range(nchunk):
            lo = c * 256
            hi = min(lo + 256, dpad)
            acc = acc + jnp.dot(dlt[:, lo:hi], w1T[dpad + lo:dpad + hi],
                                preferred_element_type=F32)
        h = jnp.maximum(acc + b1, 0.0)
        o = jnp.dot(h, w2T, preferred_element_type=F32)
        out = jnp.maximum(out, o)
    out = out + b2_ref[...]
    if not last:
        out_refs[0][0] = jnp.maximum(out, 0.0)
        return
    wposT_ref, bpos_ref, wquatT_ref, bquat_ref, wscaleT_ref, bscale_ref = head_ws
    dp = jnp.dot(out, wposT_ref[...], preferred_element_type=F32) + bpos_ref[...]
    dq = jnp.dot(out, wquatT_ref[...], preferred_element_type=F32) + bquat_ref[...]
    ds = jnp.dot(out, wscaleT_ref[...], preferred_element_type=F32) + bscale_ref[...]
    nrm = jnp.sqrt(jnp.sum(dq * dq, axis=1, keepdims=True))
    dq = dq / jnp.maximum(nrm, 1e-12)
    out_refs[0][0] = dp
    out_refs[1][0] = dq
    out_refs[2][0] = ds


def _run_edge(g, gl, last, head_params=None):
    d = g.shape[-1]
    d_out = gl['W2'].shape[0]
    x2 = jnp.sum(g * g, axis=-1)
    w1T = gl['W1'].T
    if d % 128:
        dp = d + (-d % 128)
        g = jnp.pad(g, ((0, 0), (0, 0), (0, dp - d)))
        zrows = jnp.zeros((dp - d, w1T.shape[1]), F32)
        w1T = jnp.concatenate([w1T[:d], zrows, w1T[d:], zrows], axis=0)
        d = dp
    gT = jnp.transpose(g, (0, 2, 1))
    ins = [g, gT, x2[:, :, None], x2[:, None, :],
           w1T, gl['b1'][None, :], gl['W2'].T, gl['b2'][None, :]]
    in_specs = [
        pl.BlockSpec((1, S, d), lambda b: (b, 0, 0)),
        pl.BlockSpec((1, d, S), lambda b: (b, 0, 0)),
        pl.BlockSpec((1, S, 1), lambda b: (b, 0, 0)),
        pl.BlockSpec((1, 1, S), lambda b: (b, 0, 0)),
        pl.BlockSpec((2 * d, G_HID), lambda b: (0, 0)),
        pl.BlockSpec((1, G_HID), lambda b: (0, 0)),
        pl.BlockSpec((G_HID, d_out), lambda b: (0, 0)),
        pl.BlockSpec((1, d_out), lambda b: (0, 0)),
    ]
    if not last:
        kfn = functools.partial(_edge_kernel, d_out=d_out, last=False)
        out = pl.pallas_call(
            kfn,
            grid=(B,),
            in_specs=in_specs,
            out_specs=pl.BlockSpec((1, S, d_out), lambda b: (b, 0, 0)),
            out_shape=jax.ShapeDtypeStruct((B, S, d_out), F32),
            scratch_shapes=[pltpu.VMEM((K, S, S), F32)],
            compiler_params=pltpu.CompilerParams(
                dimension_semantics=("arbitrary",)),
        )(*ins)
        return out
    hp = head_params
    ins += [hp['W_pos'].T, hp['b_pos'][None, :],
            hp['W_quat'].T, hp['b_quat'][None, :],
            hp['W_scale'].T, hp['b_scale'][None, :]]
    in_specs += [
        pl.BlockSpec((d_out, 3), lambda b: (0, 0)),
        pl.BlockSpec((1, 3), lambda b: (0, 0)),
        pl.BlockSpec((d_out, 4), lambda b: (0, 0)),
        pl.BlockSpec((1, 4), lambda b: (0, 0)),
        pl.BlockSpec((d_out, 3), lambda b: (0, 0)),
        pl.BlockSpec((1, 3), lambda b: (0, 0)),
    ]

    def kfn(g_ref, gT_ref, x2c_ref, x2r_ref, w1T_ref, b1_ref, w2T_ref, b2_ref,
            wposT, bpos, wquatT, bquat, wscaleT, bscale,
            dp_ref, dq_ref, ds_ref, ohs_ref):
        _edge_kernel(g_ref, gT_ref, x2c_ref, x2r_ref, w1T_ref, b1_ref,
                     w2T_ref, b2_ref, dp_ref, dq_ref, ds_ref, ohs_ref,
                     d_out=d_out, last=True,
                     head_ws=(wposT, bpos, wquatT, bquat, wscaleT, bscale))

    dp, dq, ds = pl.pallas_call(
        kfn,
        grid=(B,),
        in_specs=in_specs,
        out_specs=[
            pl.BlockSpec((1, S, 3), lambda b: (b, 0, 0)),
            pl.BlockSpec((1, S, 4), lambda b: (b, 0, 0)),
            pl.BlockSpec((1, S, 3), lambda b: (b, 0, 0)),
        ],
        out_shape=[
            jax.ShapeDtypeStruct((B, S, 3), F32),
            jax.ShapeDtypeStruct((B, S, 4), F32),
            jax.ShapeDtypeStruct((B, S, 3), F32),
        ],
        scratch_shapes=[pltpu.VMEM((K, S, S), F32)],
        compiler_params=pltpu.CompilerParams(
            dimension_semantics=("arbitrary",)),
    )(*ins)
    return dp, dq, ds


def kernel(positions, batch_indices, params):
    del batch_indices
    pos_tm = jnp.transpose(positions, (1, 0, 2)).reshape(S * B, 3)
    feats_tm = _run_lstm(pos_tm, params)
    feats = jnp.transpose(feats_tm.reshape(S, B, 2 * H), (1, 0, 2))
    g = jnp.concatenate([positions, feats], axis=-1)
    glayers = params['graph']
    g = _run_edge(g, glayers[0], last=False)
    g = _run_edge(g, glayers[1], last=False)
    return _run_edge(g, glayers[2], last=True, head_params=params)
